# baseline (speedup 1.0000x reference)
import functools

import jax
import jax.numpy as jnp
from jax import lax
from jax.experimental import pallas as pl
from jax.experimental.pallas import tpu as pltpu
from jax.experimental.pallas import tpu_sc as plsc

N = 10000
E = 320000
H1 = 16
H2 = 8
NC = 2
NS = 16
NW = NC * NS
NP = 10240
RPT = NP // NS
EPT = E // NW
CH = 1000
NFC = EPT // CH
EPH = E // NS
BPS = NP // NW

_sc_mesh = plsc.VectorSubcoreMesh(core_axis_name="c", subcore_axis_name="s")
_sc_params = pltpu.CompilerParams(needs_layout_passes=False,
                                  use_tc_tiling_on_sc=False)


@functools.partial(
    pl.kernel,
    out_type=jax.ShapeDtypeStruct((NP // 8, 128), jnp.float32),
    mesh=_sc_mesh,
    compiler_params=_sc_params,
    scratch_types=[
        pltpu.VMEM((EPH,), jnp.int32),
        pltpu.VMEM((NP,), jnp.float32),
        pltpu.VMEM((NS, BPS), jnp.float32),
        pltpu.VMEM((BPS,), jnp.float32),
        pltpu.VMEM((BPS // 8, 128), jnp.float32),
        pltpu.VMEM_SHARED((NS, NP), jnp.float32),
    ],
)
def _deg_dinv(ei_hbm, out_hbm, idx_v, hist_v, red_v, db_v, d2_v, stage_sh):
    cid = lax.axis_index("c")
    sid = lax.axis_index("s")
    pltpu.sync_copy(ei_hbm.at[1, pl.ds(sid * EPH, EPH)], idx_v)
    zeros = jnp.zeros((16,), jnp.float32)

    def zbody(i, carry):
        for u in range(8):
            hist_v[pl.ds(i * 128 + u * 16, 16)] = zeros
        return carry

    lax.fori_loop(0, NP // 128, zbody, 0)
    ones = jnp.ones((16,), jnp.float32)

    def hbody(i, carry):
        for u in range(5):
            vec = idx_v[pl.ds(i * 80 + u * 16, 16)]
            plsc.addupdate_scatter(hist_v, [vec], ones)
        return carry

    lax.fori_loop(0, EPH // 80, hbody, 0)
    pltpu.sync_copy(hist_v, stage_sh.at[sid])
    plsc.subcore_barrier()
    g = sid * NC + cid
    for k in range(NS):
        pltpu.sync_copy(stage_sh.at[k, pl.ds(g * BPS, BPS)], red_v.at[k])
    half = jnp.full((16,), 0.5, jnp.float32)
    threehalf = jnp.full((16,), 1.5, jnp.float32)
    magic = jnp.full((16,), 0x5F3759DF, jnp.int32)

    def dbody(i, carry):
        acc = jnp.ones((16,), jnp.float32)
        for k in range(NS):
            acc = acc + red_v[k, pl.ds(i * 16, 16)]
        yi = magic - lax.shift_right_logical(plsc.bitcast(acc, jnp.int32), 1)
        y = plsc.bitcast(yi, jnp.float32)
        hx = half * acc
        for _ in range(3):
            y = y * (threehalf - hx * y * y)
        db_v[pl.ds(i * 16, 16)] = y
        return carry

    lax.fori_loop(0, BPS // 16, dbody, 0)

    def bbody(r, carry):
        for c in range(8):
            bc = plsc.load_gather(db_v, [jnp.full((16,), r * 8 + c,
                                                  jnp.int32)])
            d2_v[r, pl.ds(c * 16, 16)] = bc
        return carry

    lax.fori_loop(0, BPS // 8, bbody, 0)
    pltpu.sync_copy(d2_v, out_hbm.at[pl.ds(g * (BPS // 8), BPS // 8)])


@functools.partial(
    pl.kernel,
    out_type=jax.ShapeDtypeStruct((NC, NP, H1), jnp.float32),
    mesh=_sc_mesh,
    compiler_params=_sc_params,
    scratch_types=[
        pltpu.VMEM((4, CH), jnp.int32),
        pltpu.VMEM((4, CH), jnp.int32),
        pltpu.VMEM((4, CH, H1), jnp.float32),
        pltpu.VMEM((RPT, H1), jnp.float32),
        pltpu.VMEM_SHARED((NP, H1), jnp.float32),
        pltpu.SemaphoreType.DMA((4,)),
        pltpu.SemaphoreType.DMA((4,)),
        pltpu.SemaphoreType.DMA((2,)),
    ],
)
def _edge_scatter(g_hbm, ei_hbm, out_hbm, sidx_v, didx_v,
                  buf_v, z_v, acc_sh, isem, gsem, ssem):
    cid = lax.axis_index("c")
    sid = lax.axis_index("s")
    wid = sid * NC + cid
    base = wid * EPT
    zeros = jnp.zeros((16,), jnp.float32)

    def zbody(i, carry):
        for u in range(8):
            z_v[i * 8 + u, :] = zeros
        return carry

    lax.fori_loop(0, RPT // 8, zbody, 0)

    idd = [None] * NFC
    gd = [None] * NFC
    sd = [None] * NFC

    def fire_idx(j):
        k = j % 4
        idd[j] = (
            pltpu.async_copy(ei_hbm.at[0, pl.ds(base + j * CH, CH)],
                             sidx_v.at[k], isem.at[k]),
            pltpu.async_copy(ei_hbm.at[1, pl.ds(base + j * CH, CH)],
                             didx_v.at[k], isem.at[k]),
        )

    def fire_gather(j):
        k = j % 4
        idd[j][0].wait()
        idd[j][1].wait()
        gd[j] = pltpu.async_copy(g_hbm.at[sidx_v.at[k]], buf_v.at[k],
                                 gsem.at[k])

    def fire_scatter(j):
        k = j % 4
        gd[j].wait()
        sd[j] = pltpu.async_copy(buf_v.at[k], acc_sh.at[didx_v.at[k]],
                                 ssem.at[j % 2], add=True)

    fire_idx(0)
    fire_idx(1)
    zd = pltpu.async_copy(z_v, acc_sh.at[pl.ds(sid * RPT, RPT)], gsem.at[3])
    fire_gather(0)
    zd.wait()
    plsc.subcore_barrier()
    for j in range(NFC):
        if j + 2 < NFC:
            if j + 2 >= 4:
                sd[j - 2].wait()
            fire_idx(j + 2)
        if j + 1 < NFC:
            fire_gather(j + 1)
        fire_scatter(j)
    for j in range(NFC - 4, NFC):
        sd[j].wait()
    plsc.subcore_barrier()
    pltpu.sync_copy(acc_sh.at[pl.ds(sid * RPT, RPT)],
                    out_hbm.at[cid, pl.ds(sid * RPT, RPT)])


NPK = NP // 8
NK = N // 8


def _tc_a_body(xg_ref, w1e_ref, d2_ref, g1_ref):
    p = jnp.dot(xg_ref[...], w1e_ref[...], preferred_element_type=jnp.float32)
    g1_ref[...] = p * d2_ref[:NK, :]


_tc_a = pl.pallas_call(
    _tc_a_body,
    out_shape=jax.ShapeDtypeStruct((NK, 128), jnp.float32),
)


def _tc_b_body(s_ref, g1_ref, d2_ref, b1_ref, w2_ref, g2_ref):
    s = s_ref[0, :NK, :] + s_ref[1, :NK, :]
    d2 = d2_ref[:NK, :]
    h1 = jnp.maximum(d2 * (s + g1_ref[...]) + b1_ref[...], 0.0)
    g2_ref[...] = jnp.dot(h1, w2_ref[...],
                          preferred_element_type=jnp.float32) * d2


_tc_b = pl.pallas_call(
    _tc_b_body,
    out_shape=jax.ShapeDtypeStruct((NK, 128), jnp.float32),
)


def _tc_c_body(s_ref, g2_ref, d2_ref, b2_ref, out_ref):
    s = s_ref[0, :NK, :] + s_ref[1, :NK, :]
    out_ref[...] = d2_ref[:NK, :] * s + g2_ref[...] * d2_ref[:NK, :] \
        + b2_ref[...]


_tc_c = pl.pallas_call(
    _tc_c_body,
    out_shape=jax.ShapeDtypeStruct((NK, 128), jnp.float32),
)


def kernel(x, edge_index, W1, b1, W2, b2):
    ei = edge_index.astype(jnp.int32)
    eye8 = jnp.eye(8, dtype=jnp.float32)
    w1e = jnp.kron(eye8, W1)
    w2bd = jnp.kron(eye8, jnp.pad(W2, ((0, 0), (0, H1 - H2))))
    b1t = jnp.tile(b1, 8)[None, :]
    b2t = jnp.tile(jnp.pad(b2, (0, H1 - H2)), 8)[None, :]

    d2p = _deg_dinv(ei)
    xg = x.reshape(NK, 8 * 128)
    g1p = _tc_a(xg, w1e, d2p)
    s1 = _edge_scatter(g1p.reshape(N, H1), ei)
    g2p = _tc_b(s1.reshape(NC, NPK, 128), g1p, d2p, b1t, w2bd)
    s2 = _edge_scatter(g2p.reshape(N, H1), ei)
    outp = _tc_c(s2.reshape(NC, NPK, 128), g2p, d2p, b2t)
    return outp.reshape(N, H1)[:, :H2]


# speedup vs baseline: 1.0036x; 1.0036x over previous
import functools

import jax
import jax.numpy as jnp
from jax import lax
from jax.experimental import pallas as pl
from jax.experimental.pallas import tpu as pltpu
from jax.experimental.pallas import tpu_sc as plsc

N = 10000
E = 320000
H1 = 16
H2 = 8
NC = 2
NS = 16
NW = NC * NS
NP = 10240
RPT = NP // NS
EPT = E // NW
CH = 1000
NFC = EPT // CH
EPH = E // NS
BPS = NP // NW

_sc_mesh = plsc.VectorSubcoreMesh(core_axis_name="c", subcore_axis_name="s")
_sc_params = pltpu.CompilerParams(needs_layout_passes=False,
                                  use_tc_tiling_on_sc=False)


@functools.partial(
    pl.kernel,
    out_type=jax.ShapeDtypeStruct((NP // 8, 128), jnp.float32),
    mesh=_sc_mesh,
    compiler_params=_sc_params,
    scratch_types=[
        pltpu.VMEM((EPH,), jnp.int32),
        pltpu.VMEM((NP,), jnp.float32),
        pltpu.VMEM((NS, BPS), jnp.float32),
        pltpu.VMEM((BPS,), jnp.float32),
        pltpu.VMEM((BPS // 8, 128), jnp.float32),
        pltpu.VMEM_SHARED((NS, NP), jnp.float32),
    ],
)
def _deg_dinv(ei_hbm, out_hbm, idx_v, hist_v, red_v, db_v, d2_v, stage_sh):
    cid = lax.axis_index("c")
    sid = lax.axis_index("s")
    pltpu.sync_copy(ei_hbm.at[1, pl.ds(sid * EPH, EPH)], idx_v)
    zeros = jnp.zeros((16,), jnp.float32)

    def zbody(i, carry):
        for u in range(8):
            hist_v[pl.ds(i * 128 + u * 16, 16)] = zeros
        return carry

    lax.fori_loop(0, NP // 128, zbody, 0)
    ones = jnp.ones((16,), jnp.float32)

    def hbody(i, carry):
        for u in range(5):
            vec = idx_v[pl.ds(i * 80 + u * 16, 16)]
            plsc.addupdate_scatter(hist_v, [vec], ones)
        return carry

    lax.fori_loop(0, EPH // 80, hbody, 0)
    pltpu.sync_copy(hist_v, stage_sh.at[sid])
    plsc.subcore_barrier()
    g = sid * NC + cid
    for k in range(NS):
        pltpu.sync_copy(stage_sh.at[k, pl.ds(g * BPS, BPS)], red_v.at[k])
    half = jnp.full((16,), 0.5, jnp.float32)
    threehalf = jnp.full((16,), 1.5, jnp.float32)
    magic = jnp.full((16,), 0x5F3759DF, jnp.int32)

    def dbody(i, carry):
        acc = jnp.ones((16,), jnp.float32)
        for k in range(NS):
            acc = acc + red_v[k, pl.ds(i * 16, 16)]
        yi = magic - lax.shift_right_logical(plsc.bitcast(acc, jnp.int32), 1)
        y = plsc.bitcast(yi, jnp.float32)
        hx = half * acc
        for _ in range(3):
            y = y * (threehalf - hx * y * y)
        db_v[pl.ds(i * 16, 16)] = y
        return carry

    lax.fori_loop(0, BPS // 16, dbody, 0)

    def bbody(r, carry):
        for c in range(8):
            bc = plsc.load_gather(db_v, [jnp.full((16,), r * 8 + c,
                                                  jnp.int32)])
            d2_v[r, pl.ds(c * 16, 16)] = bc
        return carry

    lax.fori_loop(0, BPS // 8, bbody, 0)
    pltpu.sync_copy(d2_v, out_hbm.at[pl.ds(g * (BPS // 8), BPS // 8)])


@functools.partial(
    pl.kernel,
    out_type=jax.ShapeDtypeStruct((NC, NP, H1), jnp.float32),
    mesh=_sc_mesh,
    compiler_params=_sc_params,
    scratch_types=[
        pltpu.VMEM((4, CH), jnp.int32),
        pltpu.VMEM((4, CH), jnp.int32),
        pltpu.VMEM((4, CH, H1), jnp.float32),
        pltpu.VMEM((RPT, H1), jnp.float32),
        pltpu.VMEM_SHARED((NP, H1), jnp.float32),
        pltpu.SemaphoreType.DMA((4,)),
        pltpu.SemaphoreType.DMA((4,)),
        pltpu.SemaphoreType.DMA((2,)),
    ],
)
def _edge_scatter(g_hbm, ei_hbm, out_hbm, sidx_v, didx_v,
                  buf_v, z_v, acc_sh, isem, gsem, ssem):
    cid = lax.axis_index("c")
    sid = lax.axis_index("s")
    wid = sid * NC + cid
    base = wid * EPT
    zeros = jnp.zeros((16,), jnp.float32)

    def zbody(i, carry):
        for u in range(8):
            z_v[i * 8 + u, :] = zeros
        return carry

    lax.fori_loop(0, RPT // 8, zbody, 0)

    idd = [None] * NFC
    gd = [None] * NFC
    sd = [None] * NFC

    def fire_idx(j):
        k = j % 4
        idd[j] = (
            pltpu.async_copy(ei_hbm.at[0, pl.ds(base + j * CH, CH)],
                             sidx_v.at[k], isem.at[k]),
            pltpu.async_copy(ei_hbm.at[1, pl.ds(base + j * CH, CH)],
                             didx_v.at[k], isem.at[k]),
        )

    def fire_gather(j):
        k = j % 4
        idd[j][0].wait()
        idd[j][1].wait()
        gd[j] = pltpu.async_copy(g_hbm.at[sidx_v.at[k]], buf_v.at[k],
                                 gsem.at[k])

    def fire_scatter(j):
        k = j % 4
        gd[j].wait()
        sd[j] = pltpu.async_copy(buf_v.at[k], acc_sh.at[didx_v.at[k]],
                                 ssem.at[j % 2], add=True)

    fire_idx(0)
    fire_idx(1)
    zd = pltpu.async_copy(z_v, acc_sh.at[pl.ds(sid * RPT, RPT)], gsem.at[3])
    fire_gather(0)
    zd.wait()
    plsc.subcore_barrier()
    for j in range(NFC):
        if j + 2 < NFC:
            if j + 2 >= 4:
                sd[j - 2].wait()
            fire_idx(j + 2)
        if j + 1 < NFC:
            fire_gather(j + 1)
        fire_scatter(j)
    for j in range(NFC - 4, NFC):
        sd[j].wait()
    plsc.subcore_barrier()
    pltpu.sync_copy(acc_sh.at[pl.ds(sid * RPT, RPT)],
                    out_hbm.at[cid, pl.ds(sid * RPT, RPT)])


NPK = NP // 8
NK = N // 8


def _tc_a_body(xg_ref, w1e_ref, d2_ref, g1_ref):
    p = jnp.dot(xg_ref[...].astype(jnp.float32), w1e_ref[...],
                preferred_element_type=jnp.float32)
    g1_ref[...] = p * d2_ref[:NK, :]


_tc_a = pl.pallas_call(
    _tc_a_body,
    out_shape=jax.ShapeDtypeStruct((NK, 128), jnp.float32),
)


def _tc_b_body(s_ref, g1_ref, d2_ref, b1_ref, w2_ref, g2_ref):
    s = s_ref[0, :NK, :] + s_ref[1, :NK, :]
    d2 = d2_ref[:NK, :]
    h1 = jnp.maximum(d2 * (s + g1_ref[...]) + b1_ref[...], 0.0)
    g2_ref[...] = jnp.dot(h1, w2_ref[...],
                          preferred_element_type=jnp.float32) * d2


_tc_b = pl.pallas_call(
    _tc_b_body,
    out_shape=jax.ShapeDtypeStruct((NK, 128), jnp.float32),
)


def _tc_c_body(s_ref, g2_ref, d2_ref, b2_ref, out_ref):
    s = s_ref[0, :NK, :] + s_ref[1, :NK, :]
    out_ref[...] = d2_ref[:NK, :] * s + g2_ref[...] * d2_ref[:NK, :] \
        + b2_ref[...]


_tc_c = pl.pallas_call(
    _tc_c_body,
    out_shape=jax.ShapeDtypeStruct((NK, 128), jnp.float32),
)


def kernel(x, edge_index, W1, b1, W2, b2):
    ei = edge_index.astype(jnp.int32)
    eye8 = jnp.eye(8, dtype=jnp.float32)
    w1e = jnp.kron(eye8, W1)
    w2bd = jnp.kron(eye8, jnp.pad(W2, ((0, 0), (0, H1 - H2))))
    b1t = jnp.tile(b1, 8)[None, :]
    b2t = jnp.tile(jnp.pad(b2, (0, H1 - H2)), 8)[None, :]

    d2p = _deg_dinv(ei)
    xg = x.astype(jnp.bfloat16).reshape(NK, 8 * 128)
    g1p = _tc_a(xg, w1e, d2p)
    s1 = _edge_scatter(g1p.reshape(N, H1), ei)
    g2p = _tc_b(s1.reshape(NC, NPK, 128), g1p, d2p, b1t, w2bd)
    s2 = _edge_scatter(g2p.reshape(N, H1), ei)
    outp = _tc_c(s2.reshape(NC, NPK, 128), g2p, d2p, b2t)
    return outp.reshape(N, H1)[:, :H2]


# speedup vs baseline: 1.0187x; 1.0150x over previous
import functools

import jax
import jax.numpy as jnp
from jax import lax
from jax.experimental import pallas as pl
from jax.experimental.pallas import tpu as pltpu
from jax.experimental.pallas import tpu_sc as plsc

N = 10000
E = 320000
H1 = 16
H2 = 8
NC = 2
NS = 16
NW = NC * NS
NP = 10240
RPT = NP // NS
EPT = E // NW
CH = 1000
NFC = EPT // CH
EPH = E // NS
BPS = NP // NW

_sc_mesh = plsc.VectorSubcoreMesh(core_axis_name="c", subcore_axis_name="s")
_sc_params = pltpu.CompilerParams(needs_layout_passes=False,
                                  use_tc_tiling_on_sc=False)


@functools.partial(
    pl.kernel,
    out_type=jax.ShapeDtypeStruct((NP // 8, 128), jnp.float32),
    mesh=_sc_mesh,
    compiler_params=_sc_params,
    scratch_types=[
        pltpu.VMEM((EPH,), jnp.int32),
        pltpu.VMEM((NP,), jnp.float32),
        pltpu.VMEM((NS, BPS), jnp.float32),
        pltpu.VMEM((BPS,), jnp.float32),
        pltpu.VMEM((BPS // 8, 128), jnp.float32),
        pltpu.VMEM_SHARED((NS, NP), jnp.float32),
        pltpu.SemaphoreType.DMA,
    ],
)
def _deg_dinv(ei_hbm, out_hbm, idx_v, hist_v, red_v, db_v, d2_v, stage_sh,
              sem):
    cid = lax.axis_index("c")
    sid = lax.axis_index("s")
    idxd = pltpu.async_copy(ei_hbm.at[1, pl.ds(sid * EPH, EPH)], idx_v, sem)
    zeros = jnp.zeros((16,), jnp.float32)

    def zbody(i, carry):
        for u in range(8):
            hist_v[pl.ds(i * 128 + u * 16, 16)] = zeros
        return carry

    lax.fori_loop(0, NP // 128, zbody, 0)
    idxd.wait()
    ones = jnp.ones((16,), jnp.float32)

    def hbody(i, carry):
        for u in range(5):
            vec = idx_v[pl.ds(i * 80 + u * 16, 16)]
            plsc.addupdate_scatter(hist_v, [vec], ones)
        return carry

    lax.fori_loop(0, EPH // 80, hbody, 0)
    pltpu.sync_copy(hist_v, stage_sh.at[sid])
    plsc.subcore_barrier()
    g = sid * NC + cid
    pltpu.sync_copy(stage_sh.at[:, pl.ds(g * BPS, BPS)], red_v)
    half = jnp.full((16,), 0.5, jnp.float32)
    threehalf = jnp.full((16,), 1.5, jnp.float32)
    magic = jnp.full((16,), 0x5F3759DF, jnp.int32)

    def dbody(i, carry):
        acc = jnp.ones((16,), jnp.float32)
        for k in range(NS):
            acc = acc + red_v[k, pl.ds(i * 16, 16)]
        yi = magic - lax.shift_right_logical(plsc.bitcast(acc, jnp.int32), 1)
        y = plsc.bitcast(yi, jnp.float32)
        hx = half * acc
        for _ in range(3):
            y = y * (threehalf - hx * y * y)
        db_v[pl.ds(i * 16, 16)] = y
        return carry

    lax.fori_loop(0, BPS // 16, dbody, 0)

    def bbody(r, carry):
        for c in range(8):
            bc = plsc.load_gather(db_v, [jnp.full((16,), r * 8 + c,
                                                  jnp.int32)])
            d2_v[r, pl.ds(c * 16, 16)] = bc
        return carry

    lax.fori_loop(0, BPS // 8, bbody, 0)
    pltpu.sync_copy(d2_v, out_hbm.at[pl.ds(g * (BPS // 8), BPS // 8)])


@functools.partial(
    pl.kernel,
    out_type=jax.ShapeDtypeStruct((NC, NP, H1), jnp.float32),
    mesh=_sc_mesh,
    compiler_params=_sc_params,
    scratch_types=[
        pltpu.VMEM((4, CH), jnp.int32),
        pltpu.VMEM((4, CH), jnp.int32),
        pltpu.VMEM((4, CH, H1), jnp.float32),
        pltpu.VMEM((RPT, H1), jnp.float32),
        pltpu.VMEM_SHARED((NP, H1), jnp.float32),
        pltpu.SemaphoreType.DMA((4,)),
        pltpu.SemaphoreType.DMA((4,)),
        pltpu.SemaphoreType.DMA((2,)),
    ],
)
def _edge_scatter(g_hbm, ei_hbm, out_hbm, sidx_v, didx_v,
                  buf_v, z_v, acc_sh, isem, gsem, ssem):
    cid = lax.axis_index("c")
    sid = lax.axis_index("s")
    wid = sid * NC + cid
    base = wid * EPT
    zeros = jnp.zeros((16,), jnp.float32)

    def zbody(i, carry):
        for u in range(8):
            z_v[i * 8 + u, :] = zeros
        return carry

    lax.fori_loop(0, RPT // 8, zbody, 0)

    idd = [None] * NFC
    gd = [None] * NFC
    sd = [None] * NFC

    def fire_idx(j):
        k = j % 4
        idd[j] = (
            pltpu.async_copy(ei_hbm.at[0, pl.ds(base + j * CH, CH)],
                             sidx_v.at[k], isem.at[k]),
            pltpu.async_copy(ei_hbm.at[1, pl.ds(base + j * CH, CH)],
                             didx_v.at[k], isem.at[k]),
        )

    def fire_gather(j):
        k = j % 4
        idd[j][0].wait()
        idd[j][1].wait()
        gd[j] = pltpu.async_copy(g_hbm.at[sidx_v.at[k]], buf_v.at[k],
                                 gsem.at[k])

    def fire_scatter(j):
        k = j % 4
        gd[j].wait()
        sd[j] = pltpu.async_copy(buf_v.at[k], acc_sh.at[didx_v.at[k]],
                                 ssem.at[j % 2], add=True)

    fire_idx(0)
    fire_idx(1)
    zd = pltpu.async_copy(z_v, acc_sh.at[pl.ds(sid * RPT, RPT)], gsem.at[3])
    fire_gather(0)
    zd.wait()
    plsc.subcore_barrier()
    for j in range(NFC):
        if j + 2 < NFC:
            if j + 2 >= 4:
                sd[j - 2].wait()
            fire_idx(j + 2)
        if j + 1 < NFC:
            fire_gather(j + 1)
        fire_scatter(j)
    for j in range(NFC - 4, NFC):
        sd[j].wait()
    plsc.subcore_barrier()
    pltpu.sync_copy(acc_sh.at[pl.ds(sid * RPT, RPT)],
                    out_hbm.at[cid, pl.ds(sid * RPT, RPT)])


NPK = NP // 8
NK = N // 8


def _tc_a_body(xg_ref, w1e_ref, d2_ref, g1_ref):
    p = jnp.dot(xg_ref[...], w1e_ref[...], preferred_element_type=jnp.float32)
    g1_ref[...] = p * d2_ref[:NK, :]


_tc_a = pl.pallas_call(
    _tc_a_body,
    out_shape=jax.ShapeDtypeStruct((NK, 128), jnp.float32),
)


def _tc_b_body(s_ref, g1_ref, d2_ref, b1_ref, w2_ref, g2_ref):
    s = s_ref[0, :NK, :] + s_ref[1, :NK, :]
    d2 = d2_ref[:NK, :]
    h1 = jnp.maximum(d2 * (s + g1_ref[...]) + b1_ref[...], 0.0)
    g2_ref[...] = jnp.dot(h1, w2_ref[...],
                          preferred_element_type=jnp.float32) * d2


_tc_b = pl.pallas_call(
    _tc_b_body,
    out_shape=jax.ShapeDtypeStruct((NK, 128), jnp.float32),
)


def _tc_c_body(s_ref, g2_ref, d2_ref, b2_ref, out_ref):
    s = s_ref[0, :NK, :] + s_ref[1, :NK, :]
    out_ref[...] = d2_ref[:NK, :] * s + g2_ref[...] * d2_ref[:NK, :] \
        + b2_ref[...]


_tc_c = pl.pallas_call(
    _tc_c_body,
    out_shape=jax.ShapeDtypeStruct((NK, 128), jnp.float32),
)


def kernel(x, edge_index, W1, b1, W2, b2):
    ei = edge_index.astype(jnp.int32)
    eye8 = jnp.eye(8, dtype=jnp.float32)
    w1e = jnp.kron(eye8, W1)
    w2bd = jnp.kron(eye8, jnp.pad(W2, ((0, 0), (0, H1 - H2))))
    b1t = jnp.tile(b1, 8)[None, :]
    b2t = jnp.tile(jnp.pad(b2, (0, H1 - H2)), 8)[None, :]

    d2p = _deg_dinv(ei)
    xg = x.reshape(NK, 8 * 128)
    g1p = _tc_a(xg, w1e, d2p)
    s1 = _edge_scatter(g1p.reshape(N, H1), ei)
    g2p = _tc_b(s1.reshape(NC, NPK, 128), g1p, d2p, b1t, w2bd)
    s2 = _edge_scatter(g2p.reshape(N, H1), ei)
    outp = _tc_c(s2.reshape(NC, NPK, 128), g2p, d2p, b2t)
    return outp.reshape(N, H1)[:, :H2]
